# baseline (speedup 1.0000x reference)
import jax
import jax.numpy as jnp
from jax import lax
from jax.experimental import pallas as pl
from jax.experimental.pallas import tpu as pltpu
from jax.experimental.pallas import tpu_sc as plsc

LENS_A = [1500, 900, 2100, 1100]
LENS_B = [500, 1100, 300, 900]
D = 256
N_ORIG = sum(LENS_A) + sum(LENS_B)
N = 8448

_ORDERED_LENS = [LENS_A[0], LENS_B[0], LENS_A[1], LENS_B[1],
                 LENS_A[2], LENS_B[2], LENS_A[3], LENS_B[3]]
_OFFSETS = []
_off = 0
for _l in _ORDERED_LENS:
    _OFFSETS.append(_off)
    _off += _l

_SEG_STARTS = [0, 2000, 4000, 6400]
_NPAT = 5
_GROUP = 4
NWORKERS = 32
ROWS_PER_W = N // NWORKERS
GROUPS_PER_W = ROWS_PER_W // _GROUP


def _seed_kernel(out_ref):
    r = jax.lax.broadcasted_iota(jnp.int32, (_NPAT * _GROUP, 1), 0) // _GROUP
    k = jax.lax.broadcasted_iota(jnp.int32, (1, N), 1)
    sid_k = jnp.zeros(k.shape, jnp.int32)
    for b in _SEG_STARTS[1:]:
        sid_k = sid_k + (k >= b).astype(jnp.int32)
    out_ref[...] = (r == sid_k) & (r < 4) & (k < N_ORIG)


def _seg_of(r):
    p = jnp.int32(0)
    for b in _SEG_STARTS[1:] + [N_ORIG]:
        p = p + (r >= b).astype(jnp.int32)
    return p


def _sc_replicate_kernel(seed_hbm, out_hbm, pat_v, sem):
    wid = lax.axis_index("s") * 2 + lax.axis_index("c")
    base = wid * ROWS_PER_W
    p_lo = _seg_of(base)
    p_hi = _seg_of(base + ROWS_PER_W - 1)
    pltpu.sync_copy(seed_hbm.at[pl.ds(p_lo * _GROUP, _GROUP), :], pat_v.at[0])
    pltpu.sync_copy(seed_hbm.at[pl.ds(p_hi * _GROUP, _GROUP), :], pat_v.at[1])
    copies = []
    for g in range(GROUPS_PER_W):
        r = base + g * _GROUP
        sel = (_seg_of(r) > p_lo).astype(jnp.int32)
        c = pltpu.make_async_copy(
            pat_v.at[sel],
            out_hbm.at[pl.ds(r, _GROUP), :],
            sem,
        )
        c.start()
        copies.append(c)
    for c in copies:
        c.wait()


def _pack_kernel(a0, a1, a2, a3, b0, b1, b2, b3, out_ref):
    ins = [a0, b0, a1, b1, a2, b2, a3, b3]
    for ref, off, l in zip(ins, _OFFSETS, _ORDERED_LENS):
        out_ref[0, off:off + l, :] = ref[0]
    out_ref[0, N_ORIG:N, :] = jnp.zeros((N - N_ORIG, D), jnp.float32)


def kernel(a0, a1, a2, a3, b0, b1, b2, b3):
    seed = pl.pallas_call(
        _seed_kernel,
        out_shape=jax.ShapeDtypeStruct((_NPAT * _GROUP, N), jnp.bool_),
    )()

    sc_mesh = plsc.VectorSubcoreMesh(core_axis_name="c", subcore_axis_name="s")
    mask = pl.kernel(
        _sc_replicate_kernel,
        out_type=jax.ShapeDtypeStruct((N, N), jnp.bool_),
        mesh=sc_mesh,
        scratch_types=[
            pltpu.VMEM((2, _GROUP, N), jnp.bool_),
            pltpu.SemaphoreType.DMA,
        ],
    )(seed)

    packed = pl.pallas_call(
        _pack_kernel,
        out_shape=jax.ShapeDtypeStruct((1, N, D), jnp.float32),
    )(a0, a1, a2, a3, b0, b1, b2, b3)
    return packed, mask


# speedup vs baseline: 1.1045x; 1.1045x over previous
import jax
import jax.numpy as jnp
from jax.experimental import pallas as pl

LENS_A = [1500, 900, 2100, 1100]
LENS_B = [500, 1100, 300, 900]
D = 256
N_ORIG = sum(LENS_A) + sum(LENS_B)
N = 8448

_ORDERED_LENS = [LENS_A[0], LENS_B[0], LENS_A[1], LENS_B[1],
                 LENS_A[2], LENS_B[2], LENS_A[3], LENS_B[3]]
_OFFSETS = []
_off = 0
for _l in _ORDERED_LENS:
    _OFFSETS.append(_off)
    _off += _l

_SEG_STARTS = [0, 2000, 4000, 6400]

TILE_R = 384
NTILES = N // TILE_R
PACK_TILE = 2 * N // NTILES

_IN_ORDER = [0, 4, 1, 5, 2, 6, 3, 7]


def _fused_kernel(a0, a1, a2, a3, b0, b1, b2, b3, mask_ref, packed_ref):
    t = pl.program_id(0)

    q = jax.lax.broadcasted_iota(jnp.int32, (TILE_R, 1), 0) + t * TILE_R
    k = jax.lax.broadcasted_iota(jnp.int32, (1, N), 1)

    def seg_id(p):
        s = jnp.zeros(p.shape, jnp.int32)
        for b in _SEG_STARTS[1:]:
            s = s + (p >= b).astype(jnp.int32)
        return s

    mask_ref[...] = (seg_id(q) == seg_id(k)) & (q < N_ORIG) & (k < N_ORIG)

    ins = [a0, a1, a2, a3, b0, b1, b2, b3]
    for step in range(NTILES):
        lo, hi = PACK_TILE * step, PACK_TILE * (step + 1)

        def _copies(lo=lo, hi=hi):
            for idx, off, l in zip(_IN_ORDER, _OFFSETS, _ORDERED_LENS):
                s0, s1 = max(lo, 2 * off), min(hi, 2 * (off + l))
                if s0 < s1:
                    packed_ref[s0 - lo:s1 - lo, :] = (
                        ins[idx][s0 - 2 * off:s1 - 2 * off, :])
            z0, z1 = max(lo, 2 * N_ORIG), hi
            if z0 < z1:
                packed_ref[z0 - lo:z1 - lo, :] = jnp.zeros(
                    (z1 - z0, 128), jnp.float32)

        pl.when(t == step)(_copies)


def kernel(a0, a1, a2, a3, b0, b1, b2, b3):
    views = [jnp.reshape(x, (2 * x.shape[1], 128))
             for x in (a0, a1, a2, a3, b0, b1, b2, b3)]
    full_specs = [
        pl.BlockSpec((2 * x.shape[1], 128), lambda t: (0, 0))
        for x in (a0, a1, a2, a3, b0, b1, b2, b3)
    ]
    mask, packed2d = pl.pallas_call(
        _fused_kernel,
        grid=(NTILES,),
        in_specs=full_specs,
        out_specs=(
            pl.BlockSpec((TILE_R, N), lambda t: (t, 0)),
            pl.BlockSpec((PACK_TILE, 128), lambda t: (t, 0)),
        ),
        out_shape=(
            jax.ShapeDtypeStruct((N, N), jnp.bool_),
            jax.ShapeDtypeStruct((2 * N, 128), jnp.float32),
        ),
    )(*views)
    return jnp.reshape(packed2d, (1, N, D)), mask
